# baseline (speedup 1.0000x reference)
import functools

import jax
import jax.numpy as jnp
from jax import lax
from jax.experimental import pallas as pl
from jax.experimental.pallas import tpu as pltpu
from jax.experimental.pallas import tpu_sc as plsc

SRC_LEN, BSZ, H = 4096, 16, 256
M = 32768
C = 4096
NROWS = SRC_LEN * BSZ

NC, NS, L = 2, 16, 16
NW = NC * NS

CPT = C // NW
SPT = SRC_LEN // NW
RPT = NROWS // NW
CHUNK = 128
MCHUNK = 64
NEG = -3.4e38
IMIN = -(2**31)


@functools.cache
def _mesh():
    return plsc.VectorSubcoreMesh(core_axis_name="c", subcore_axis_name="s",
                                  num_cores=NC, num_subcores=NS)


def _at_rows(tbl, idx_ref):
    return tbl.at[idx_ref]


def _wid():
    return lax.axis_index("s") * NC + lax.axis_index("c")


def _phys_rows(p):
    return ((p & (SRC_LEN - 1)) << 4) | (p >> 12)


def _lane():
    return lax.iota(jnp.int32, L)


def _elem(tbl_vmem, idx, n, scr):
    start = jnp.minimum(idx, n - L)
    scr[pl.ds(0, L)] = tbl_vmem[pl.ds(start, L)]
    return scr[pl.ds(idx - start, L)][0]


def _search(tbl_vmem, bound, n, scr):
    def body(_, carry):
        lo, hi = carry
        mid = (lo + hi) >> 1
        v = _elem(tbl_vmem, mid, n, scr)
        pred = jnp.logical_and(mid < hi, v < bound)
        return (jnp.where(pred, mid + 1, lo), jnp.where(pred, hi, mid))

    lo, _ = lax.fori_loop(0, 16, body, (jnp.int32(0), jnp.int32(n)))
    return lo


def _gather_segmax_body(m_flat, mp_hbm, cid_hbm, g_out, pooled_out,
                        cid_vmem, mp_chunk, idx_chunk, rows, pooled_loc, scr, sem):
    w = _wid()
    pltpu.sync_copy(cid_hbm, cid_vmem)

    def initrow(i, _):
        for v in range(H // L):
            pooled_loc[i, pl.ds(v * L, L)] = jnp.full((L,), NEG, jnp.float32)
        return 0

    lax.fori_loop(0, CPT, initrow, 0)

    c_lo = w * CPT
    lo = _search(cid_vmem, c_lo, M, scr)
    hi = _search(cid_vmem, c_lo + CPT, M, scr)
    k0 = lo >> 7
    k1 = (hi + CHUNK - 1) >> 7

    def chunk_body(k, _):
        base = k * CHUNK
        pltpu.sync_copy(mp_hbm.at[pl.ds(base, CHUNK)], mp_chunk)
        for g in range(CHUNK // L):
            p = mp_chunk[pl.ds(g * L, L)]
            idx_chunk[pl.ds(g * L, L)] = _phys_rows(p)
        pltpu.async_copy(_at_rows(m_flat, idx_chunk), rows, sem).wait()
        pltpu.sync_copy(rows, g_out.at[pl.ds(base, CHUNK)])

        def seg_group(g, _):
            cv = cid_vmem[pl.ds(base + g * L, L)]
            for l in range(L):
                t = cv[l] - c_lo

                @pl.when(jnp.logical_and(t >= 0, t < CPT))
                def _(t=t, g=g, l=l):
                    i = g * L + l
                    for v in range(H // L):
                        sl = pl.ds(v * L, L)
                        pooled_loc[t, sl] = jnp.maximum(pooled_loc[t, sl],
                                                        rows[i, sl])

            return 0

        lax.fori_loop(0, CHUNK // L, seg_group, 0)
        return 0

    lax.fori_loop(k0, k1, chunk_body, 0)
    pltpu.sync_copy(pooled_loc, pooled_out.at[pl.ds(c_lo, CPT)])


def _merge_scatter_body(m_flat, mp_hbm, cid_hbm, gw_hbm, pw_hbm, out,
                        mp_vmem, cid_vmem, work_i, work_c, work_r,
                        gw_buf, pw_buf, scr, sem):
    w = _wid()
    pltpu.sync_copy(mp_hbm, mp_vmem)
    pltpu.sync_copy(cid_hbm, cid_vmem)
    pltpu.sync_copy(m_flat.at[pl.ds(w * RPT, RPT)], out.at[pl.ds(w * RPT, RPT)])

    lane = _lane()
    base = jnp.int32(0)
    for b in range(BSZ):
        lo_b = _search(mp_vmem, b * SRC_LEN + w * SPT, M, scr)
        hi_b = _search(mp_vmem, b * SRC_LEN + w * SPT + SPT, M, scr)
        len_b = hi_b - lo_b
        p0 = _elem(mp_vmem, lo_b, M, scr)
        c0 = _elem(cid_vmem, lo_b, M, scr)
        for j in range(SPT // L):
            @pl.when(j * L < len_b)
            def _(lo_b=lo_b, hi_b=hi_b, base=base, j=j, p0=p0, c0=c0):
                start = jnp.minimum(lo_b + j * L, M - L)
                idx = start + lane
                valid = jnp.logical_and(idx >= lo_b, idx < hi_b)
                p = jnp.where(valid, mp_vmem[pl.ds(start, L)], p0)
                cv = jnp.where(valid, cid_vmem[pl.ds(start, L)], c0)
                src = jnp.where(valid, idx, lo_b)
                off = base + j * L
                row = off >> 6
                col = off & (MCHUNK - 1)
                work_i[row, pl.ds(col, L)] = src
                work_c[row, pl.ds(col, L)] = cv
                work_r[row, pl.ds(col, L)] = _phys_rows(p)

        base = base + ((len_b + L - 1) & ~(L - 1))

    total = base
    nchunks = (total + MCHUNK - 1) >> 6

    @pl.when(total > 0)
    def _():
        fi = jnp.full((L,), work_i[0, pl.ds(0, L)][0], jnp.int32)
        fc = jnp.full((L,), work_c[0, pl.ds(0, L)][0], jnp.int32)
        fr = jnp.full((L,), work_r[0, pl.ds(0, L)][0], jnp.int32)
        for g in range(3):
            off = total + g * L
            row = off >> 6
            col = off & (MCHUNK - 1)
            work_i[row, pl.ds(col, L)] = fi
            work_c[row, pl.ds(col, L)] = fc
            work_r[row, pl.ds(col, L)] = fr

    def merge_body(k, _):
        pltpu.async_copy(_at_rows(gw_hbm, work_i.at[k]), gw_buf, sem).wait()
        pltpu.async_copy(_at_rows(pw_hbm, work_c.at[k]), pw_buf, sem).wait()

        def row_body(j, _):
            for v in range(H // L):
                sl = pl.ds(v * L, L)
                x = gw_buf[j, sl] + pw_buf[j, sl]
                e = jnp.exp(-2.0 * jnp.abs(x))
                t = (1.0 - e) / (1.0 + e)
                gw_buf[j, sl] = jnp.where(x < 0.0, -t, t)
            return 0

        lax.fori_loop(0, MCHUNK, row_body, 0)
        pltpu.async_copy(gw_buf, _at_rows(out, work_r.at[k]), sem).wait()
        return 0

    lax.fori_loop(0, nchunks, merge_body, 0)


def _matmul_bias_body(x_ref, w_ref, b_ref, o_ref):
    o_ref[...] = jnp.dot(x_ref[...], w_ref[...],
                         preferred_element_type=jnp.float32) + b_ref[...]


def _matmul_body(x_ref, w_ref, o_ref):
    o_ref[...] = jnp.dot(x_ref[...], w_ref[...],
                         preferred_element_type=jnp.float32)


def kernel(m_bank, W, b, mention_pos, cluster_ids):
    src_len, bsz, h = m_bank.shape
    m_flat = m_bank.reshape(src_len * bsz, h)
    w1 = W[:h]
    w2 = W[h:]
    b2d = b.reshape(1, h)

    gather_segmax = pl.kernel(
        _gather_segmax_body,
        out_type=(
            jax.ShapeDtypeStruct((M, H), jnp.float32),
            jax.ShapeDtypeStruct((C, H), jnp.float32),
        ),
        mesh=_mesh(),
        scratch_types=[
            pltpu.VMEM((M,), jnp.int32),
            pltpu.VMEM((CHUNK,), jnp.int32),
            pltpu.VMEM((CHUNK,), jnp.int32),
            pltpu.VMEM((CHUNK, H), jnp.float32),
            pltpu.VMEM((CPT, H), jnp.float32),
            pltpu.VMEM((2 * L,), jnp.int32),
            pltpu.SemaphoreType.DMA,
        ],
    )
    g_rows, pooled = gather_segmax(m_flat, mention_pos, cluster_ids)

    blk = 256
    pw = pl.pallas_call(
        _matmul_bias_body,
        out_shape=jax.ShapeDtypeStruct((C, h), jnp.float32),
        grid=(C // blk,),
        in_specs=[
            pl.BlockSpec((blk, h), lambda i: (i, 0)),
            pl.BlockSpec((h, h), lambda i: (0, 0)),
            pl.BlockSpec((1, h), lambda i: (0, 0)),
        ],
        out_specs=pl.BlockSpec((blk, h), lambda i: (i, 0)),
    )(pooled, w2, b2d)

    gw = pl.pallas_call(
        _matmul_body,
        out_shape=jax.ShapeDtypeStruct((M, h), jnp.float32),
        grid=(M // blk,),
        in_specs=[
            pl.BlockSpec((blk, h), lambda i: (i, 0)),
            pl.BlockSpec((h, h), lambda i: (0, 0)),
        ],
        out_specs=pl.BlockSpec((blk, h), lambda i: (i, 0)),
    )(g_rows, w1)

    merge_scatter = pl.kernel(
        _merge_scatter_body,
        out_type=jax.ShapeDtypeStruct((NROWS, H), jnp.float32),
        mesh=_mesh(),
        scratch_types=[
            pltpu.VMEM((M,), jnp.int32),
            pltpu.VMEM((M,), jnp.int32),
            pltpu.VMEM((34, MCHUNK), jnp.int32),
            pltpu.VMEM((34, MCHUNK), jnp.int32),
            pltpu.VMEM((34, MCHUNK), jnp.int32),
            pltpu.VMEM((MCHUNK, H), jnp.float32),
            pltpu.VMEM((MCHUNK, H), jnp.float32),
            pltpu.VMEM((2 * L,), jnp.int32),
            pltpu.SemaphoreType.DMA,
        ],
    )
    out_flat = merge_scatter(m_flat, mention_pos, cluster_ids, gw, pw)
    return out_flat.reshape(src_len, bsz, h)


# speedup vs baseline: 1.0079x; 1.0079x over previous
import functools

import jax
import jax.numpy as jnp
from jax import lax
from jax.experimental import pallas as pl
from jax.experimental.pallas import tpu as pltpu
from jax.experimental.pallas import tpu_sc as plsc

SRC_LEN, BSZ, H = 4096, 16, 256
M = 32768
C = 4096
NROWS = SRC_LEN * BSZ

NC, NS, L = 2, 16, 16
NW = NC * NS

CPT = C // NW
SPT = SRC_LEN // NW
RPT = NROWS // NW
CHUNK = 128
MCHUNK = 64
NEG = -3.4e38
IMIN = -(2**31)


@functools.cache
def _mesh():
    return plsc.VectorSubcoreMesh(core_axis_name="c", subcore_axis_name="s",
                                  num_cores=NC, num_subcores=NS)


def _at_rows(tbl, idx_ref):
    return tbl.at[idx_ref]


def _wid():
    return lax.axis_index("s") * NC + lax.axis_index("c")


def _phys_rows(p):
    return ((p & (SRC_LEN - 1)) << 4) | (p >> 12)


def _lane():
    return lax.iota(jnp.int32, L)


def _elem(tbl_vmem, idx, n, scr):
    start = jnp.minimum(idx, n - L)
    scr[pl.ds(0, L)] = tbl_vmem[pl.ds(start, L)]
    return scr[pl.ds(idx - start, L)][0]


def _search(tbl_vmem, bound, n, scr):
    def body(_, carry):
        lo, hi = carry
        mid = (lo + hi) >> 1
        v = _elem(tbl_vmem, mid, n, scr)
        pred = jnp.logical_and(mid < hi, v < bound)
        return (jnp.where(pred, mid + 1, lo), jnp.where(pred, hi, mid))

    lo, _ = lax.fori_loop(0, 16, body, (jnp.int32(0), jnp.int32(n)))
    return lo


def _gather_segmax_body(m_flat, mp_hbm, cid_hbm, g_out, pooled_out,
                        cid_vmem, mp_chunk, idx_chunk, rows, pooled_loc, scr, sem):
    w = _wid()
    pltpu.sync_copy(cid_hbm, cid_vmem)

    def initrow(i, _):
        for v in range(H // L):
            pooled_loc[i, pl.ds(v * L, L)] = jnp.full((L,), NEG, jnp.float32)
        return 0

    lax.fori_loop(0, CPT, initrow, 0)

    c_lo = w * CPT
    lo = _search(cid_vmem, c_lo, M, scr)
    hi = _search(cid_vmem, c_lo + CPT, M, scr)
    k0 = lo >> 7
    k1 = (hi + CHUNK - 1) >> 7

    def chunk_body(k, _):
        base = k * CHUNK
        pltpu.sync_copy(mp_hbm.at[pl.ds(base, CHUNK)], mp_chunk)
        for g in range(CHUNK // L):
            p = mp_chunk[pl.ds(g * L, L)]
            idx_chunk[pl.ds(g * L, L)] = _phys_rows(p)
        pltpu.async_copy(_at_rows(m_flat, idx_chunk), rows, sem).wait()
        pltpu.sync_copy(rows, g_out.at[pl.ds(base, CHUNK)])

        def seg_group(g, _):
            cv = cid_vmem[pl.ds(base + g * L, L)]
            for l in range(L):
                t = cv[l] - c_lo

                @pl.when(jnp.logical_and(t >= 0, t < CPT))
                def _(t=t, g=g, l=l):
                    i = g * L + l
                    for v in range(H // L):
                        sl = pl.ds(v * L, L)
                        pooled_loc[t, sl] = jnp.maximum(pooled_loc[t, sl],
                                                        rows[i, sl])

            return 0

        lax.fori_loop(0, CHUNK // L, seg_group, 0)
        return 0

    lax.fori_loop(k0, k1, chunk_body, 0)
    pltpu.sync_copy(pooled_loc, pooled_out.at[pl.ds(c_lo, CPT)])


def _scatter_body(m_flat, mp_hbm, y_hbm, out,
                  mp_vmem, work_i, work_r, y_buf, scr, sem):
    w = _wid()
    pltpu.sync_copy(mp_hbm, mp_vmem)
    pltpu.sync_copy(m_flat.at[pl.ds(w * RPT, RPT)], out.at[pl.ds(w * RPT, RPT)])

    lane = _lane()
    base = jnp.int32(0)
    for b in range(BSZ):
        lo_b = _search(mp_vmem, b * SRC_LEN + w * SPT, M, scr)
        hi_b = _search(mp_vmem, b * SRC_LEN + w * SPT + SPT, M, scr)
        len_b = hi_b - lo_b
        p0 = _elem(mp_vmem, lo_b, M, scr)
        for j in range(SPT // L):
            @pl.when(j * L < len_b)
            def _(lo_b=lo_b, hi_b=hi_b, base=base, j=j, p0=p0):
                start = jnp.minimum(lo_b + j * L, M - L)
                idx = start + lane
                valid = jnp.logical_and(idx >= lo_b, idx < hi_b)
                p = jnp.where(valid, mp_vmem[pl.ds(start, L)], p0)
                src = jnp.where(valid, idx, lo_b)
                off = base + j * L
                row = off >> 6
                col = off & (MCHUNK - 1)
                work_i[row, pl.ds(col, L)] = src
                work_r[row, pl.ds(col, L)] = _phys_rows(p)

        base = base + ((len_b + L - 1) & ~(L - 1))

    total = base
    nchunks = (total + MCHUNK - 1) >> 6

    @pl.when(total > 0)
    def _():
        fi = jnp.full((L,), work_i[0, pl.ds(0, L)][0], jnp.int32)
        fr = jnp.full((L,), work_r[0, pl.ds(0, L)][0], jnp.int32)
        for g in range(3):
            off = total + g * L
            row = off >> 6
            col = off & (MCHUNK - 1)
            work_i[row, pl.ds(col, L)] = fi
            work_r[row, pl.ds(col, L)] = fr

    def merge_body(k, _):
        pltpu.async_copy(_at_rows(y_hbm, work_i.at[k]), y_buf, sem).wait()
        pltpu.async_copy(y_buf, _at_rows(out, work_r.at[k]), sem).wait()
        return 0

    lax.fori_loop(0, nchunks, merge_body, 0)


EPT = M // NW
ECH = 128


def _expand_body(pw_hbm, cid2_hbm, e_out, cidv, buf, sem):
    w = _wid()
    pltpu.sync_copy(cid2_hbm.at[pl.ds(w * (EPT // ECH), EPT // ECH)], cidv)

    def chunk(j, _):
        pltpu.async_copy(_at_rows(pw_hbm, cidv.at[j]), buf, sem).wait()
        pltpu.sync_copy(buf, e_out.at[pl.ds(w * EPT + j * ECH, ECH)])
        return 0

    lax.fori_loop(0, EPT // ECH, chunk, 0)


def _matmul_bias_body(x_ref, w_ref, b_ref, o_ref):
    o_ref[...] = jnp.dot(x_ref[...], w_ref[...],
                         preferred_element_type=jnp.float32) + b_ref[...]


def _merge_matmul_body(g_ref, w_ref, e_ref, o_ref):
    o_ref[...] = jnp.tanh(jnp.dot(g_ref[...], w_ref[...],
                                  preferred_element_type=jnp.float32)
                          + e_ref[...])


def kernel(m_bank, W, b, mention_pos, cluster_ids):
    src_len, bsz, h = m_bank.shape
    m_flat = m_bank.reshape(src_len * bsz, h)
    w1 = W[:h]
    w2 = W[h:]
    b2d = b.reshape(1, h)

    gather_segmax = pl.kernel(
        _gather_segmax_body,
        out_type=(
            jax.ShapeDtypeStruct((M, H), jnp.float32),
            jax.ShapeDtypeStruct((C, H), jnp.float32),
        ),
        mesh=_mesh(),
        scratch_types=[
            pltpu.VMEM((M,), jnp.int32),
            pltpu.VMEM((CHUNK,), jnp.int32),
            pltpu.VMEM((CHUNK,), jnp.int32),
            pltpu.VMEM((CHUNK, H), jnp.float32),
            pltpu.VMEM((CPT, H), jnp.float32),
            pltpu.VMEM((2 * L,), jnp.int32),
            pltpu.SemaphoreType.DMA,
        ],
    )
    g_rows, pooled = gather_segmax(m_flat, mention_pos, cluster_ids)

    blk = 256
    pw = pl.pallas_call(
        _matmul_bias_body,
        out_shape=jax.ShapeDtypeStruct((C, h), jnp.float32),
        grid=(C // blk,),
        in_specs=[
            pl.BlockSpec((blk, h), lambda i: (i, 0)),
            pl.BlockSpec((h, h), lambda i: (0, 0)),
            pl.BlockSpec((1, h), lambda i: (0, 0)),
        ],
        out_specs=pl.BlockSpec((blk, h), lambda i: (i, 0)),
    )(pooled, w2, b2d)

    expand = pl.kernel(
        _expand_body,
        out_type=jax.ShapeDtypeStruct((M, H), jnp.float32),
        mesh=_mesh(),
        scratch_types=[
            pltpu.VMEM((M // NW // ECH, ECH), jnp.int32),
            pltpu.VMEM((ECH, H), jnp.float32),
            pltpu.SemaphoreType.DMA,
        ],
    )
    cid2 = cluster_ids.reshape(M // ECH, ECH)
    e_rows = expand(pw, cid2)

    y = pl.pallas_call(
        _merge_matmul_body,
        out_shape=jax.ShapeDtypeStruct((M, h), jnp.float32),
        grid=(M // blk,),
        in_specs=[
            pl.BlockSpec((blk, h), lambda i: (i, 0)),
            pl.BlockSpec((h, h), lambda i: (0, 0)),
            pl.BlockSpec((blk, h), lambda i: (i, 0)),
        ],
        out_specs=pl.BlockSpec((blk, h), lambda i: (i, 0)),
    )(g_rows, w1, e_rows)

    scatter = pl.kernel(
        _scatter_body,
        out_type=jax.ShapeDtypeStruct((NROWS, H), jnp.float32),
        mesh=_mesh(),
        scratch_types=[
            pltpu.VMEM((M,), jnp.int32),
            pltpu.VMEM((34, MCHUNK), jnp.int32),
            pltpu.VMEM((34, MCHUNK), jnp.int32),
            pltpu.VMEM((MCHUNK, H), jnp.float32),
            pltpu.VMEM((2 * L,), jnp.int32),
            pltpu.SemaphoreType.DMA,
        ],
    )
    out_flat = scatter(m_flat, mention_pos, y)
    return out_flat.reshape(src_len, bsz, h)


# speedup vs baseline: 5.5128x; 5.4697x over previous
import functools

import jax
import jax.numpy as jnp
from jax import lax
from jax.experimental import pallas as pl
from jax.experimental.pallas import tpu as pltpu
from jax.experimental.pallas import tpu_sc as plsc

SRC_LEN, BSZ, H = 4096, 16, 256
M = 32768
C = 4096
NROWS = SRC_LEN * BSZ

NC, NS, L = 2, 16, 16
NW = NC * NS

CPT = C // NW
SPT = SRC_LEN // NW
RPT = NROWS // NW
CHUNK = 128
MCHUNK = 64
NEG = -3.4e38
IMIN = -(2**31)


@functools.cache
def _mesh():
    return plsc.VectorSubcoreMesh(core_axis_name="c", subcore_axis_name="s",
                                  num_cores=NC, num_subcores=NS)


def _at_rows(tbl, idx_ref):
    return tbl.at[idx_ref]


def _wid():
    return lax.axis_index("s") * NC + lax.axis_index("c")


def _phys_rows(p):
    return ((p & (SRC_LEN - 1)) << 4) | (p >> 12)


def _lane():
    return lax.iota(jnp.int32, L)


def _elem(tbl_vmem, idx, n, scr):
    start = jnp.minimum(idx, n - L)
    scr[pl.ds(0, L)] = tbl_vmem[pl.ds(start, L)]
    return scr[pl.ds(idx - start, L)][0]


def _search(tbl_vmem, bound, n, scr):
    def body(_, carry):
        lo, hi = carry
        mid = (lo + hi) >> 1
        v = _elem(tbl_vmem, mid, n, scr)
        pred = jnp.logical_and(mid < hi, v < bound)
        return (jnp.where(pred, mid + 1, lo), jnp.where(pred, hi, mid))

    lo, _ = lax.fori_loop(0, 16, body, (jnp.int32(0), jnp.int32(n)))
    return lo


def _gather_segmax_body(m_flat, mp_hbm, cid_hbm, g_out, pooled_out,
                        cid_vmem, mp_chunk, idx_chunk, rows, pooled_loc, scr, sem):
    w = _wid()
    pltpu.sync_copy(cid_hbm, cid_vmem)

    def initrow(i, _):
        for v in range(H // L):
            pooled_loc[i, pl.ds(v * L, L)] = jnp.full((L,), NEG, jnp.float32)
        return 0

    lax.fori_loop(0, CPT, initrow, 0)

    c_lo = w * CPT
    lo = _search(cid_vmem, c_lo, M, scr)
    hi = _search(cid_vmem, c_lo + CPT, M, scr)
    k0 = lo >> 7
    k1 = (hi + CHUNK - 1) >> 7

    def chunk_body(k, _):
        base = k * CHUNK
        pltpu.sync_copy(mp_hbm.at[pl.ds(base, CHUNK)], mp_chunk)
        for g in range(CHUNK // L):
            p = mp_chunk[pl.ds(g * L, L)]
            idx_chunk[pl.ds(g * L, L)] = _phys_rows(p)
        pltpu.async_copy(_at_rows(m_flat, idx_chunk), rows, sem).wait()
        pltpu.sync_copy(rows, g_out.at[pl.ds(base, CHUNK)])

        def seg_group(g, _):
            cv = cid_vmem[pl.ds(base + g * L, L)]
            for l in range(L):
                t = cv[l] - c_lo

                @pl.when(jnp.logical_and(t >= 0, t < CPT))
                def _(t=t, g=g, l=l):
                    i = g * L + l
                    for v in range(H // L):
                        sl = pl.ds(v * L, L)
                        pooled_loc[t, sl] = jnp.maximum(pooled_loc[t, sl],
                                                        rows[i, sl])

            return 0

        lax.fori_loop(0, CHUNK // L, seg_group, 0)
        return 0

    lax.fori_loop(k0, k1, chunk_body, 0)
    pltpu.sync_copy(pooled_loc, pooled_out.at[pl.ds(c_lo, CPT)])


CCH = 128
NCOPY = RPT // CCH


def _scatter_body(m_flat, mp_hbm, y_hbm, out,
                  mp_vmem, work_i, work_r, y_buf, cp0, cp1, scr,
                  sem, sem_i0, sem_i1, sem_o0, sem_o1):
    w = _wid()
    pltpu.sync_copy(mp_hbm, mp_vmem)
    cbufs = (cp0, cp1)
    sins = (sem_i0, sem_i1)
    souts = (sem_o0, sem_o1)
    out_descs = [None, None]
    for c in range(NCOPY):
        bi = c % 2
        if out_descs[bi] is not None:
            out_descs[bi].wait()
        off = w * RPT + c * CCH
        pltpu.async_copy(m_flat.at[pl.ds(off, CCH)], cbufs[bi], sins[bi]).wait()
        out_descs[bi] = pltpu.async_copy(cbufs[bi], out.at[pl.ds(off, CCH)],
                                         souts[bi])
    out_descs[0].wait()
    out_descs[1].wait()

    lane = _lane()
    base = jnp.int32(0)
    for b in range(BSZ):
        lo_b = _search(mp_vmem, b * SRC_LEN + w * SPT, M, scr)
        hi_b = _search(mp_vmem, b * SRC_LEN + w * SPT + SPT, M, scr)
        len_b = hi_b - lo_b
        p0 = _elem(mp_vmem, lo_b, M, scr)
        for j in range(SPT // L):
            @pl.when(j * L < len_b)
            def _(lo_b=lo_b, hi_b=hi_b, base=base, j=j, p0=p0):
                start = jnp.minimum(lo_b + j * L, M - L)
                idx = start + lane
                valid = jnp.logical_and(idx >= lo_b, idx < hi_b)
                p = jnp.where(valid, mp_vmem[pl.ds(start, L)], p0)
                src = jnp.where(valid, idx, lo_b)
                off = base + j * L
                row = off >> 6
                col = off & (MCHUNK - 1)
                work_i[row, pl.ds(col, L)] = src
                work_r[row, pl.ds(col, L)] = _phys_rows(p)

        base = base + ((len_b + L - 1) & ~(L - 1))

    total = base
    nchunks = (total + MCHUNK - 1) >> 6

    @pl.when(total > 0)
    def _():
        fi = jnp.full((L,), work_i[0, pl.ds(0, L)][0], jnp.int32)
        fr = jnp.full((L,), work_r[0, pl.ds(0, L)][0], jnp.int32)
        for g in range(3):
            off = total + g * L
            row = off >> 6
            col = off & (MCHUNK - 1)
            work_i[row, pl.ds(col, L)] = fi
            work_r[row, pl.ds(col, L)] = fr

    def merge_body(k, _):
        pltpu.async_copy(_at_rows(y_hbm, work_i.at[k]), y_buf, sem).wait()
        pltpu.async_copy(y_buf, _at_rows(out, work_r.at[k]), sem).wait()
        return 0

    lax.fori_loop(0, nchunks, merge_body, 0)


EPT = M // NW
ECH = 128


def _expand_body(pw_hbm, cid2_hbm, e_out, cidv, buf, sem):
    w = _wid()
    pltpu.sync_copy(cid2_hbm.at[pl.ds(w * (EPT // ECH), EPT // ECH)], cidv)

    def chunk(j, _):
        pltpu.async_copy(_at_rows(pw_hbm, cidv.at[j]), buf, sem).wait()
        pltpu.sync_copy(buf, e_out.at[pl.ds(w * EPT + j * ECH, ECH)])
        return 0

    lax.fori_loop(0, EPT // ECH, chunk, 0)


def _matmul_bias_body(x_ref, w_ref, b_ref, o_ref):
    o_ref[...] = jnp.dot(x_ref[...], w_ref[...],
                         preferred_element_type=jnp.float32) + b_ref[...]


def _merge_matmul_body(g_ref, w_ref, e_ref, o_ref):
    o_ref[...] = jnp.tanh(jnp.dot(g_ref[...], w_ref[...],
                                  preferred_element_type=jnp.float32)
                          + e_ref[...])


def kernel(m_bank, W, b, mention_pos, cluster_ids):
    src_len, bsz, h = m_bank.shape
    m_flat = m_bank.reshape(src_len * bsz, h)
    w1 = W[:h]
    w2 = W[h:]
    b2d = b.reshape(1, h)

    gather_segmax = pl.kernel(
        _gather_segmax_body,
        out_type=(
            jax.ShapeDtypeStruct((M, H), jnp.float32),
            jax.ShapeDtypeStruct((C, H), jnp.float32),
        ),
        mesh=_mesh(),
        scratch_types=[
            pltpu.VMEM((M,), jnp.int32),
            pltpu.VMEM((CHUNK,), jnp.int32),
            pltpu.VMEM((CHUNK,), jnp.int32),
            pltpu.VMEM((CHUNK, H), jnp.float32),
            pltpu.VMEM((CPT, H), jnp.float32),
            pltpu.VMEM((2 * L,), jnp.int32),
            pltpu.SemaphoreType.DMA,
        ],
    )
    g_rows, pooled = gather_segmax(m_flat, mention_pos, cluster_ids)

    blk = 256
    pw = pl.pallas_call(
        _matmul_bias_body,
        out_shape=jax.ShapeDtypeStruct((C, h), jnp.float32),
        grid=(C // blk,),
        in_specs=[
            pl.BlockSpec((blk, h), lambda i: (i, 0)),
            pl.BlockSpec((h, h), lambda i: (0, 0)),
            pl.BlockSpec((1, h), lambda i: (0, 0)),
        ],
        out_specs=pl.BlockSpec((blk, h), lambda i: (i, 0)),
    )(pooled, w2, b2d)

    expand = pl.kernel(
        _expand_body,
        out_type=jax.ShapeDtypeStruct((M, H), jnp.float32),
        mesh=_mesh(),
        scratch_types=[
            pltpu.VMEM((M // NW // ECH, ECH), jnp.int32),
            pltpu.VMEM((ECH, H), jnp.float32),
            pltpu.SemaphoreType.DMA,
        ],
    )
    cid2 = cluster_ids.reshape(M // ECH, ECH)
    e_rows = expand(pw, cid2)

    y = pl.pallas_call(
        _merge_matmul_body,
        out_shape=jax.ShapeDtypeStruct((M, h), jnp.float32),
        grid=(M // blk,),
        in_specs=[
            pl.BlockSpec((blk, h), lambda i: (i, 0)),
            pl.BlockSpec((h, h), lambda i: (0, 0)),
            pl.BlockSpec((blk, h), lambda i: (i, 0)),
        ],
        out_specs=pl.BlockSpec((blk, h), lambda i: (i, 0)),
    )(g_rows, w1, e_rows)

    scatter = pl.kernel(
        _scatter_body,
        out_type=jax.ShapeDtypeStruct((NROWS, H), jnp.float32),
        mesh=_mesh(),
        scratch_types=[
            pltpu.VMEM((M,), jnp.int32),
            pltpu.VMEM((34, MCHUNK), jnp.int32),
            pltpu.VMEM((34, MCHUNK), jnp.int32),
            pltpu.VMEM((MCHUNK, H), jnp.float32),
            pltpu.VMEM((CCH, H), jnp.float32),
            pltpu.VMEM((CCH, H), jnp.float32),
            pltpu.VMEM((2 * L,), jnp.int32),
            pltpu.SemaphoreType.DMA,
            pltpu.SemaphoreType.DMA,
            pltpu.SemaphoreType.DMA,
            pltpu.SemaphoreType.DMA,
            pltpu.SemaphoreType.DMA,
        ],
    )
    out_flat = scatter(m_flat, mention_pos, y)
    return out_flat.reshape(src_len, bsz, h)
